# Initial kernel scaffold; baseline (speedup 1.0000x reference)
import jax
import jax.numpy as jnp
from jax.experimental import pallas as pl


def kernel(char_ids, buffer_idx, stack_idx, char_table, W, b):
    raise NotImplementedError("write your pallas kernel here")


# speedup vs baseline: 26.1319x; 26.1319x over previous
import functools

import jax
import jax.numpy as jnp
from jax import lax
from jax.experimental import pallas as pl
from jax.experimental.pallas import tpu as pltpu
from jax.experimental.pallas import tpu_sc as plsc

B = 16384
N_WORDS = 327680
H = 50
NSLOT = 6
NCHAR = 5
NCOL = NSLOT * 3
CID_PAD = 8

NW = 32
B_PER_W = B // NW
PAIRS_PER_W = B_PER_W * NSLOT
CHUNK = 128
NCHUNK = PAIRS_PER_W // CHUNK
GROUPS = B_PER_W // 16


def _fold_table_kernel(ct_ref, wr_ref, br_ref, out_ref):
    out_ref[...] = (
        jnp.dot(ct_ref[...], wr_ref[...], preferred_element_type=jnp.float32)
        + br_ref[...]
    )


def _build_fold_table(char_table, W, b):
    w_r = W.reshape(NSLOT, H, 3).transpose(1, 0, 2).reshape(H, NCOL)
    ct_p = jnp.zeros((512, 64), jnp.float32).at[:500, :H].set(char_table)
    wr_p = jnp.zeros((64, 128), jnp.float32).at[:H, :NCOL].set(w_r)
    br = jnp.tile(b, NSLOT) / (NSLOT * NCHAR)
    br_p = jnp.zeros((1, 128), jnp.float32).at[0, :NCOL].set(br)
    t_full = pl.pallas_call(
        _fold_table_kernel,
        out_shape=jax.ShapeDtypeStruct((512, 128), jnp.float32),
    )(ct_p, wr_p, br_p)
    return t_full[:500, :NCOL].reshape(-1)


def _sc_kernel(idx_hbm, cid_hbm, t_hbm, out_hbm, idx_v, cid_v, t_v, out_v, sem):
    nc = 2
    wid = lax.axis_index("s") * nc + lax.axis_index("c")

    pltpu.sync_copy(idx_hbm.at[pl.ds(wid * NCHUNK, NCHUNK)], idx_v)

    copies = []
    for q in range(NCHUNK):
        copies.append(
            pltpu.async_copy(
                cid_hbm.at[idx_v.at[q]],
                cid_v.at[pl.ds(q * CHUNK, CHUNK)],
                sem,
            )
        )
    pltpu.sync_copy(t_hbm, t_v)
    for c in copies:
        c.wait()

    lanes = lax.iota(jnp.int32, 16)

    def body(g, _):
        accs = [jnp.zeros((16,), jnp.float32) for _ in range(3)]
        for j in range(NSLOT):
            rowvec = lanes * NSLOT + (g * (16 * NSLOT) + j)
            for c in range(NCHAR):
                cid = plsc.load_gather(cid_v, [rowvec, jnp.full((16,), c, jnp.int32)])
                taddr = cid * NCOL
                for k in range(3):
                    accs[k] = accs[k] + plsc.load_gather(t_v, [taddr + (j * 3 + k)])
        m = jnp.maximum(accs[0], jnp.maximum(accs[1], accs[2]))
        for k in range(3):
            out_v[k, pl.ds(g * 16, 16)] = jnp.exp(accs[k] - m)
        return 0

    lax.fori_loop(0, GROUPS, body, 0)

    pltpu.sync_copy(out_v, out_hbm.at[:, pl.ds(wid * B_PER_W, B_PER_W)])


@jax.jit
def kernel(char_ids, buffer_idx, stack_idx, char_table, W, b):
    t_tab = _build_fold_table(char_table, W, b)
    cid_p = jnp.zeros((N_WORDS, CID_PAD), jnp.int32).at[:, :NCHAR].set(
        char_ids.astype(jnp.int32)
    )
    idx_r = (
        jnp.concatenate(
            [buffer_idx.astype(jnp.int32), stack_idx.astype(jnp.int32)], axis=1
        ).reshape(NW * NCHUNK, CHUNK)
    )

    mesh = plsc.VectorSubcoreMesh(core_axis_name="c", subcore_axis_name="s")
    run = functools.partial(
        pl.kernel,
        mesh=mesh,
        out_type=jax.ShapeDtypeStruct((3, B), jnp.float32),
        scratch_types=[
            pltpu.VMEM((NCHUNK, CHUNK), jnp.int32),
            pltpu.VMEM((PAIRS_PER_W, CID_PAD), jnp.int32),
            pltpu.VMEM((500 * NCOL,), jnp.float32),
            pltpu.VMEM((3, B_PER_W), jnp.float32),
            pltpu.SemaphoreType.DMA,
        ],
        compiler_params=pltpu.CompilerParams(
            needs_layout_passes=False, use_tc_tiling_on_sc=False
        ),
    )(_sc_kernel)
    out3 = run(idx_r, cid_p, t_tab)
    return out3.T
